# baseline (speedup 1.0000x reference)
import functools

import jax
import jax.numpy as jnp
from jax import lax
from jax.experimental import pallas as pl
from jax.experimental.pallas import tpu as pltpu
from jax.experimental.pallas import tpu_sc as plsc

N = 10000
NP = 10240
E = 320000
H = 6
F = 128
D = 768
ROWW = 144
TILE = 512
NT = NP // TILE
NSUB = 16
NCORE = 2
EPS = E // NSUB
K = 128
NCHUNK = EPS // K
RPS = NP // NSUB
NQ = 5
QROWS = NP // NQ
ACCR = QROWS + 8
RPQ = QROWS // NSUB
STAG = 4000
PECAP = EPS + NQ * 2 * K


def _prep_body(x_ref, w_ref, sk_ref, asrc_ref, adst_ref,
               projS_ref, sdst_ref, skout_ref):
    xt = x_ref[...]
    sdst_rows = []
    for h in range(H):
        wh = w_ref[:, h * F:(h + 1) * F]
        ph = jnp.dot(xt, wh, preferred_element_type=jnp.float32)
        projS_ref[h, :, 0:F] = ph
        ssrc = jnp.sum(ph * asrc_ref[h, :][None, :], axis=1, keepdims=True)
        projS_ref[h, :, F:ROWW] = jnp.concatenate(
            [ssrc, jnp.zeros((TILE, ROWW - F - 1), jnp.float32)], axis=1)
        sdst_rows.append(jnp.sum(ph * adst_ref[h, :][None, :], axis=1))
    sdst_rows += [jnp.zeros((TILE,), jnp.float32)] * 2
    sdst_ref[...] = jnp.stack(sdst_rows, axis=0)
    skout_ref[...] = jnp.dot(xt, sk_ref[...], preferred_element_type=jnp.float32)


def _prep_call(x, W, skipW, asrc_pad, adst_pad, interpret=False):
    return pl.pallas_call(
        _prep_body,
        grid=(NT,),
        in_specs=[
            pl.BlockSpec((TILE, D), lambda i: (i, 0)),
            pl.BlockSpec((D, D), lambda i: (0, 0)),
            pl.BlockSpec((D, D), lambda i: (0, 0)),
            pl.BlockSpec((8, F), lambda i: (0, 0)),
            pl.BlockSpec((8, F), lambda i: (0, 0)),
        ],
        out_specs=[
            pl.BlockSpec((H, TILE, ROWW), lambda i: (0, i, 0)),
            pl.BlockSpec((8, TILE), lambda i: (0, i)),
            pl.BlockSpec((TILE, D), lambda i: (i, 0)),
        ],
        out_shape=[
            jax.ShapeDtypeStruct((H, NP, ROWW), jnp.float32),
            jax.ShapeDtypeStruct((8, NP), jnp.float32),
            jax.ShapeDtypeStruct((NP, D), jnp.float32),
        ],
        interpret=interpret,
    )(x, W, skipW, asrc_pad, adst_pad)


def _epi_cat_body(u_ref, sk_ref, b_ref, out_ref):
    for h in range(H):
        u = u_ref[h, :, 0:F]
        dn = u_ref[h, :, F:F + 1]
        t = u / (dn + 1e-16) + sk_ref[:, h * F:(h + 1) * F] \
            + b_ref[0, h * F:(h + 1) * F][None, :]
        out_ref[:, h * F:(h + 1) * F] = jnp.where(t > 0, t, jnp.exp(jnp.minimum(t, 0.0)) - 1.0)


def _epi_mean_body(u_ref, sk_ref, b_ref, out_ref):
    acc = jnp.zeros((TILE, F), jnp.float32)
    for h in range(H):
        u = u_ref[h, :, 0:F]
        dn = u_ref[h, :, F:F + 1]
        acc = acc + u / (dn + 1e-16) + sk_ref[:, h * F:(h + 1) * F]
    out_ref[...] = acc * (1.0 / H) + b_ref[0, :][None, :]


def _epi_call(U, sk, b, mean, interpret=False):
    body = _epi_mean_body if mean else _epi_cat_body
    fout = F if mean else D
    return pl.pallas_call(
        body,
        grid=(NT,),
        in_specs=[
            pl.BlockSpec((H, TILE, ROWW), lambda i: (0, i, 0)),
            pl.BlockSpec((TILE, D), lambda i: (i, 0)),
            pl.BlockSpec((1, fout), lambda i: (0, 0)),
        ],
        out_specs=pl.BlockSpec((TILE, fout), lambda i: (i, 0)),
        out_shape=jax.ShapeDtypeStruct((NP, fout), jnp.float32),
        interpret=interpret,
    )(U, sk, b.reshape(1, fout))


def _edge_body(projS, sdst, srcs, dsts, u_out,
               idxA, idxB, dqbuf, rowsA, rowsB, zbuf, stag_s, stag_d,
               pe_src, pe_dst, sdst_buf, accA, semA, semB):
    c = lax.axis_index("c")
    s = lax.axis_index("s")
    ebase = s * EPS
    zero16 = jnp.zeros((16,), jnp.float32)
    c128 = jnp.full((16,), F, jnp.int32)
    i16 = lax.iota(jnp.int32, 16)
    for e in range(8):
        for r in range(ROWW // 16):
            zbuf[e, pl.ds(r * 16, 16)] = zero16

    def count_stage(st, carry):
        soff = pl.multiple_of(ebase + st * STAG, 8)
        pltpu.sync_copy(dsts.at[pl.ds(soff, STAG)], stag_d)

        def cbody(u, cnts):
            dv = stag_d[pl.ds(pl.multiple_of(u * 16, 16), 16)]
            bk = lax.shift_right_logical(dv, 11)
            return tuple(
                cnts[q] + jnp.max(jnp.cumsum(jnp.where(bk == q, 1, 0)))
                for q in range(NQ))
        return lax.fori_loop(0, STAG // 16, cbody, carry)

    cnts = lax.fori_loop(0, EPS // STAG, count_stage,
                         (jnp.int32(0),) * NQ)
    cps = [((cnt + 2 * K - 1) // (2 * K)) * (2 * K) for cnt in cnts]
    bases = [jnp.int32(0)]
    for q in range(NQ - 1):
        bases.append(bases[-1] + cps[q])

    def place_stage(st, carry):
        soff = pl.multiple_of(ebase + st * STAG, 8)
        pltpu.sync_copy(srcs.at[pl.ds(soff, STAG)], stag_s)
        pltpu.sync_copy(dsts.at[pl.ds(soff, STAG)], stag_d)

        def pbody(u, offs):
            uoff = pl.multiple_of(u * 16, 16)
            sv = stag_s[pl.ds(uoff, 16)]
            dv = stag_d[pl.ds(uoff, 16)]
            bk = lax.shift_right_logical(dv, 11)
            new_offs = []
            for q in range(NQ):
                m = bk == q
                cum = jnp.cumsum(jnp.where(m, 1, 0))
                pos = bases[q] + offs[q] + cum - 1
                plsc.store_scatter(pe_src, [pos], sv, mask=m)
                plsc.store_scatter(pe_dst, [pos], dv, mask=m)
                new_offs.append(offs[q] + jnp.max(cum))
            return tuple(new_offs)
        return lax.fori_loop(0, STAG // 16, pbody, carry)

    lax.fori_loop(0, EPS // STAG, place_stage, (jnp.int32(0),) * NQ)

    for q in range(NQ):
        for w in range(2 * K // 16):
            pos = bases[q] + cnts[q] + w * 16 + i16
            m = pos < bases[q] + cps[q]
            plsc.store_scatter(pe_src, [pos], jnp.zeros((16,), jnp.int32),
                               mask=m)
            plsc.store_scatter(pe_dst, [pos], jnp.full((16,), NP - 1,
                                                       jnp.int32), mask=m)
    steps = [cp // (2 * K) for cp in cps]

    def build_fire(idxbuf, rowsbuf, sem, qb, ch, hoff):
        boff = qb + ch * K
        for g in range(K // 16):
            sv = pe_src[pl.ds(boff + g * 16, 16)]
            idxbuf[pl.ds(g * 16, 16)] = sv + hoff
        return pltpu.async_copy(projS.at[idxbuf], rowsbuf, sem)

    def compute_scatter(rowsbuf, qb, qoff, ch):
        boff = qb + ch * K
        for g in range(K // 16):
            eidx = i16 + g * 16
            ssrc = plsc.load_gather(rowsbuf, [eidx, c128])
            dv = pe_dst[pl.ds(boff + g * 16, 16)]
            sdv = plsc.load_gather(sdst_buf, [dv])
            ev = ssrc + sdv
            ev = jnp.where(ev >= 0, ev, ev * 0.2)
            ex = jnp.exp(ev)
            plsc.store_scatter(rowsbuf, [eidx, c128], ex)
            dvq = dv - qoff
            dvq = jnp.where((dvq >= 0) & (dvq < QROWS), dvq,
                            jnp.full((16,), QROWS, jnp.int32))
            dqbuf[pl.ds(g * 16, 16)] = dvq
            for i in range(16):
                bro = lax.gather(
                    ex, jnp.full((16, 1), i, jnp.int32),
                    lax.GatherDimensionNumbers(
                        offset_dims=(), collapsed_slice_dims=(0,),
                        start_index_map=(0,)),
                    slice_sizes=(1,),
                    mode=lax.GatherScatterMode.PROMISE_IN_BOUNDS)
                e_abs = g * 16 + i
                for r in range(F // 16):
                    rowsbuf[e_abs, pl.ds(r * 16, 16)] = \
                        rowsbuf[e_abs, pl.ds(r * 16, 16)] * bro
        pltpu.sync_copy(rowsbuf, accA.at[dqbuf], add=True)

    def pass_body(t, carry0):
        q = t % NQ
        h = (t // NQ) * NCORE + c
        qoff = pl.multiple_of(q * QROWS, 8)
        hoff = pl.multiple_of(h * NP, 8)
        qb = bases[0]
        nsteps = steps[0]
        for qq in range(1, NQ):
            qb = jnp.where(q == qq, bases[qq], qb)
            nsteps = jnp.where(q == qq, steps[qq], nsteps)
        qb = pl.multiple_of(qb, 16)
        for k in range(RPQ // 8):
            pltpu.sync_copy(zbuf, accA.at[pl.ds(s * RPQ + k * 8, 8)])

        @pl.when(s == 0)
        def _zero_dump():
            pltpu.sync_copy(zbuf, accA.at[pl.ds(QROWS, 8)])

        pltpu.sync_copy(sdst.at[pl.ds(hoff, NP)], sdst_buf)
        plsc.subcore_barrier()

        @pl.when(nsteps > 0)
        def _pipeline():
            build_fire(idxA, rowsA, semA, qb, 0, hoff)

            def step(jj, carry):
                pltpu.make_async_copy(projS.at[idxA], rowsA, semA).wait()
                build_fire(idxB, rowsB, semB, qb, 2 * jj + 1, hoff)
                compute_scatter(rowsA, qb, qoff, 2 * jj)
                pltpu.make_async_copy(projS.at[idxB], rowsB, semB).wait()

                @pl.when(jj + 1 < nsteps)
                def _prefetch():
                    build_fire(idxA, rowsA, semA, qb, 2 * jj + 2, hoff)
                compute_scatter(rowsB, qb, qoff, 2 * jj + 1)
                return carry
            lax.fori_loop(0, nsteps, step, 0)
        plsc.subcore_barrier()
        pltpu.sync_copy(accA.at[pl.ds(s * RPQ, RPQ)],
                        u_out.at[pl.ds(hoff + qoff + s * RPQ, RPQ)])
        plsc.subcore_barrier()
        return carry0

    lax.fori_loop(0, 3 * NQ, pass_body, 0)


@functools.cache
def _edge_call():
    return functools.partial(
        pl.kernel,
        mesh=plsc.VectorSubcoreMesh(core_axis_name="c", subcore_axis_name="s"),
        compiler_params=pltpu.CompilerParams(needs_layout_passes=False,
                                             use_tc_tiling_on_sc=False),
        out_type=jax.ShapeDtypeStruct((H * NP, ROWW), jnp.float32),
        scratch_types=[
            pltpu.VMEM((K,), jnp.int32),
            pltpu.VMEM((K,), jnp.int32),
            pltpu.VMEM((K,), jnp.int32),
            pltpu.VMEM((K, ROWW), jnp.float32),
            pltpu.VMEM((K, ROWW), jnp.float32),
            pltpu.VMEM((8, ROWW), jnp.float32),
            pltpu.VMEM((STAG,), jnp.int32),
            pltpu.VMEM((STAG,), jnp.int32),
            pltpu.VMEM((PECAP,), jnp.int32),
            pltpu.VMEM((PECAP,), jnp.int32),
            pltpu.VMEM((NP,), jnp.float32),
            pltpu.VMEM_SHARED((ACCR, ROWW), jnp.float32),
            pltpu.SemaphoreType.DMA,
            pltpu.SemaphoreType.DMA,
        ],
    )(_edge_body)


def kernel(x, edge_index, W0, a_src0, a_dst0, skip0, b0,
           W1, a_src1, a_dst1, skip1, b1,
           W2, a_src2, a_dst2, skip2, b2):
    ei = edge_index.astype(jnp.int32)
    srcs, dsts = ei[0], ei[1]
    pad2 = jnp.zeros((2, F), jnp.float32)
    h = jnp.concatenate([x, jnp.zeros((NP - N, D), jnp.float32)], axis=0)
    layers = [(W0, a_src0, a_dst0, skip0, b0, False),
              (W1, a_src1, a_dst1, skip1, b1, False),
              (W2, a_src2, a_dst2, skip2, b2, True)]
    for (W, a_s, a_d, sk, b, mean) in layers:
        asrc_pad = jnp.concatenate([a_s, pad2], axis=0)
        adst_pad = jnp.concatenate([a_d, pad2], axis=0)
        projS, sdst, skout = _prep_call(h, W, sk, asrc_pad, adst_pad)
        U = _edge_call()(projS.reshape(H * NP, ROWW), sdst.reshape(8 * NP),
                         srcs, dsts)
        h = _epi_call(U.reshape(H, NP, ROWW), skout, b, mean)
    return h[:N]


# speedup vs baseline: 1.1071x; 1.1071x over previous
import functools

import jax
import jax.numpy as jnp
from jax import lax
from jax.experimental import pallas as pl
from jax.experimental.pallas import tpu as pltpu
from jax.experimental.pallas import tpu_sc as plsc

N = 10000
NP = 10240
E = 320000
H = 6
F = 128
D = 768
ROWW = 144
TILE = 512
NT = NP // TILE
NSUB = 16
NCORE = 2
EPS = E // NSUB
K = 80
NCHUNK = EPS // K
RPS = NP // NSUB
NQ = 5
QROWS = NP // NQ
ACCR = QROWS + 8
RPQ = QROWS // NSUB
STAG = 4000
NBUF = 3
PECAP = EPS + NQ * NBUF * K


def _prep_body(x_ref, w_ref, sk_ref, asrc_ref, adst_ref,
               projS_ref, sdst_ref, skout_ref):
    xt = x_ref[...]
    sdst_rows = []
    for h in range(H):
        wh = w_ref[:, h * F:(h + 1) * F]
        ph = jnp.dot(xt, wh, preferred_element_type=jnp.float32)
        projS_ref[h, :, 0:F] = ph
        ssrc = jnp.sum(ph * asrc_ref[h, :][None, :], axis=1, keepdims=True)
        projS_ref[h, :, F:ROWW] = jnp.concatenate(
            [ssrc, jnp.zeros((TILE, ROWW - F - 1), jnp.float32)], axis=1)
        sdst_rows.append(jnp.sum(ph * adst_ref[h, :][None, :], axis=1))
    sdst_rows += [jnp.zeros((TILE,), jnp.float32)] * 2
    sdst_ref[...] = jnp.stack(sdst_rows, axis=0)
    skout_ref[...] = jnp.dot(xt, sk_ref[...], preferred_element_type=jnp.float32)


def _prep_call(x, W, skipW, asrc_pad, adst_pad, interpret=False):
    return pl.pallas_call(
        _prep_body,
        grid=(NT,),
        in_specs=[
            pl.BlockSpec((TILE, D), lambda i: (i, 0)),
            pl.BlockSpec((D, D), lambda i: (0, 0)),
            pl.BlockSpec((D, D), lambda i: (0, 0)),
            pl.BlockSpec((8, F), lambda i: (0, 0)),
            pl.BlockSpec((8, F), lambda i: (0, 0)),
        ],
        out_specs=[
            pl.BlockSpec((H, TILE, ROWW), lambda i: (0, i, 0)),
            pl.BlockSpec((8, TILE), lambda i: (0, i)),
            pl.BlockSpec((TILE, D), lambda i: (i, 0)),
        ],
        out_shape=[
            jax.ShapeDtypeStruct((H, NP, ROWW), jnp.float32),
            jax.ShapeDtypeStruct((8, NP), jnp.float32),
            jax.ShapeDtypeStruct((NP, D), jnp.float32),
        ],
        interpret=interpret,
    )(x, W, skipW, asrc_pad, adst_pad)


def _epi_cat_body(u_ref, sk_ref, b_ref, out_ref):
    for h in range(H):
        u = u_ref[h, :, 0:F]
        dn = u_ref[h, :, F:F + 1]
        t = u / (dn + 1e-16) + sk_ref[:, h * F:(h + 1) * F] \
            + b_ref[0, h * F:(h + 1) * F][None, :]
        out_ref[:, h * F:(h + 1) * F] = jnp.where(t > 0, t, jnp.exp(jnp.minimum(t, 0.0)) - 1.0)


def _epi_mean_body(u_ref, sk_ref, b_ref, out_ref):
    acc = jnp.zeros((TILE, F), jnp.float32)
    for h in range(H):
        u = u_ref[h, :, 0:F]
        dn = u_ref[h, :, F:F + 1]
        acc = acc + u / (dn + 1e-16) + sk_ref[:, h * F:(h + 1) * F]
    out_ref[...] = acc * (1.0 / H) + b_ref[0, :][None, :]


def _epi_call(U, sk, b, mean, interpret=False):
    body = _epi_mean_body if mean else _epi_cat_body
    fout = F if mean else D
    return pl.pallas_call(
        body,
        grid=(NT,),
        in_specs=[
            pl.BlockSpec((H, TILE, ROWW), lambda i: (0, i, 0)),
            pl.BlockSpec((TILE, D), lambda i: (i, 0)),
            pl.BlockSpec((1, fout), lambda i: (0, 0)),
        ],
        out_specs=pl.BlockSpec((TILE, fout), lambda i: (i, 0)),
        out_shape=jax.ShapeDtypeStruct((NP, fout), jnp.float32),
        interpret=interpret,
    )(U, sk, b.reshape(1, fout))


def _edge_body(projS, sdst, srcs, dsts, u_out,
               idxA, idxB, idxC, dqA, dqB, dqC, rowsA, rowsB, rowsC,
               zbuf, stag_s, stag_d, pe_src, pe_dst, sdst_buf, accA,
               gsA, gsB, gsC, ssA, ssB, ssC):
    c = lax.axis_index("c")
    s = lax.axis_index("s")
    ebase = s * EPS
    zero16 = jnp.zeros((16,), jnp.float32)
    c128 = jnp.full((16,), F, jnp.int32)
    i16 = lax.iota(jnp.int32, 16)
    for e in range(8):
        for r in range(ROWW // 16):
            zbuf[e, pl.ds(r * 16, 16)] = zero16

    def count_stage(st, carry):
        soff = pl.multiple_of(ebase + st * STAG, 8)
        pltpu.sync_copy(dsts.at[pl.ds(soff, STAG)], stag_d)

        def cbody(u, cnts):
            dv = stag_d[pl.ds(pl.multiple_of(u * 16, 16), 16)]
            bk = lax.shift_right_logical(dv, 11)
            return tuple(
                cnts[q] + jnp.max(jnp.cumsum(jnp.where(bk == q, 1, 0)))
                for q in range(NQ))
        return lax.fori_loop(0, STAG // 16, cbody, carry)

    cnts = lax.fori_loop(0, EPS // STAG, count_stage,
                         (jnp.int32(0),) * NQ)
    cps = [((cnt + NBUF * K - 1) // (NBUF * K)) * (NBUF * K) for cnt in cnts]
    bases = [jnp.int32(0)]
    for q in range(NQ - 1):
        bases.append(bases[-1] + cps[q])

    def place_stage(st, carry):
        soff = pl.multiple_of(ebase + st * STAG, 8)
        pltpu.sync_copy(srcs.at[pl.ds(soff, STAG)], stag_s)
        pltpu.sync_copy(dsts.at[pl.ds(soff, STAG)], stag_d)

        def pbody(u, offs):
            uoff = pl.multiple_of(u * 16, 16)
            sv = stag_s[pl.ds(uoff, 16)]
            dv = stag_d[pl.ds(uoff, 16)]
            bk = lax.shift_right_logical(dv, 11)
            new_offs = []
            for q in range(NQ):
                m = bk == q
                cum = jnp.cumsum(jnp.where(m, 1, 0))
                pos = bases[q] + offs[q] + cum - 1
                plsc.store_scatter(pe_src, [pos], sv, mask=m)
                plsc.store_scatter(pe_dst, [pos], dv, mask=m)
                new_offs.append(offs[q] + jnp.max(cum))
            return tuple(new_offs)
        return lax.fori_loop(0, STAG // 16, pbody, carry)

    lax.fori_loop(0, EPS // STAG, place_stage, (jnp.int32(0),) * NQ)

    for q in range(NQ):
        for w in range(NBUF * K // 16):
            pos = bases[q] + cnts[q] + w * 16 + i16
            m = pos < bases[q] + cps[q]
            plsc.store_scatter(pe_src, [pos], jnp.zeros((16,), jnp.int32),
                               mask=m)
            plsc.store_scatter(pe_dst, [pos], jnp.full((16,), NP - 1,
                                                       jnp.int32), mask=m)
    steps = [cp // (NBUF * K) for cp in cps]

    def build_fire(idxbuf, rowsbuf, sem, qb, ch, hoff):
        boff = qb + ch * K
        for g in range(K // 16):
            sv = pe_src[pl.ds(boff + g * 16, 16)]
            idxbuf[pl.ds(g * 16, 16)] = sv + hoff
        return pltpu.async_copy(projS.at[idxbuf], rowsbuf, sem)

    def compute_scatter(rowsbuf, dqbuf, ssem, qb, qoff, ch):
        boff = qb + ch * K
        for g in range(K // 16):
            eidx = i16 + g * 16
            ssrc = plsc.load_gather(rowsbuf, [eidx, c128])
            dv = pe_dst[pl.ds(boff + g * 16, 16)]
            sdv = plsc.load_gather(sdst_buf, [dv])
            ev = ssrc + sdv
            ev = jnp.where(ev >= 0, ev, ev * 0.2)
            ex = jnp.exp(ev)
            plsc.store_scatter(rowsbuf, [eidx, c128], ex)
            dvq = dv - qoff
            dvq = jnp.where((dvq >= 0) & (dvq < QROWS), dvq,
                            jnp.full((16,), QROWS, jnp.int32))
            dqbuf[pl.ds(g * 16, 16)] = dvq
            for i in range(16):
                bro = lax.gather(
                    ex, jnp.full((16, 1), i, jnp.int32),
                    lax.GatherDimensionNumbers(
                        offset_dims=(), collapsed_slice_dims=(0,),
                        start_index_map=(0,)),
                    slice_sizes=(1,),
                    mode=lax.GatherScatterMode.PROMISE_IN_BOUNDS)
                e_abs = g * 16 + i
                for r in range(F // 16):
                    rowsbuf[e_abs, pl.ds(r * 16, 16)] = \
                        rowsbuf[e_abs, pl.ds(r * 16, 16)] * bro
        pltpu.async_copy(rowsbuf, accA.at[dqbuf], ssem, add=True)

    def pass_body(t, carry0):
        q = t % NQ
        h = (t // NQ) * NCORE + c
        qoff = pl.multiple_of(q * QROWS, 8)
        hoff = pl.multiple_of(h * NP, 8)
        qb = bases[0]
        nsteps = steps[0]
        for qq in range(1, NQ):
            qb = jnp.where(q == qq, bases[qq], qb)
            nsteps = jnp.where(q == qq, steps[qq], nsteps)
        qb = pl.multiple_of(qb, 16)
        for k in range(RPQ // 8):
            pltpu.sync_copy(zbuf, accA.at[pl.ds(s * RPQ + k * 8, 8)])

        @pl.when(s == 0)
        def _zero_dump():
            pltpu.sync_copy(zbuf, accA.at[pl.ds(QROWS, 8)])

        pltpu.sync_copy(sdst.at[pl.ds(hoff, NP)], sdst_buf)
        plsc.subcore_barrier()


        @pl.when(nsteps > 0)
        def _pipeline():
            build_fire(idxA, rowsA, gsA, qb, 0, hoff)
            build_fire(idxB, rowsB, gsB, qb, 1, hoff)

            def step(jj, carry):
                m0 = 3 * jj
                pltpu.make_async_copy(projS.at[idxA], rowsA, gsA).wait()
                compute_scatter(rowsA, dqA, ssA, qb, qoff, m0)

                @pl.when(jj > 0)
                def _wsc():
                    pltpu.make_async_copy(rowsC, accA.at[dqC], ssC).wait()
                build_fire(idxC, rowsC, gsC, qb, m0 + 2, hoff)
                pltpu.make_async_copy(projS.at[idxB], rowsB, gsB).wait()
                compute_scatter(rowsB, dqB, ssB, qb, qoff, m0 + 1)
                pltpu.make_async_copy(rowsA, accA.at[dqA], ssA).wait()

                @pl.when(jj + 1 < nsteps)
                def _fga():
                    build_fire(idxA, rowsA, gsA, qb, m0 + 3, hoff)
                pltpu.make_async_copy(projS.at[idxC], rowsC, gsC).wait()
                compute_scatter(rowsC, dqC, ssC, qb, qoff, m0 + 2)
                pltpu.make_async_copy(rowsB, accA.at[dqB], ssB).wait()

                @pl.when(jj + 1 < nsteps)
                def _fgb():
                    build_fire(idxB, rowsB, gsB, qb, m0 + 4, hoff)
                return carry
            lax.fori_loop(0, nsteps, step, 0)
            pltpu.make_async_copy(rowsC, accA.at[dqC], ssC).wait()
        plsc.subcore_barrier()
        pltpu.sync_copy(accA.at[pl.ds(s * RPQ, RPQ)],
                        u_out.at[pl.ds(hoff + qoff + s * RPQ, RPQ)])
        plsc.subcore_barrier()
        return carry0

    lax.fori_loop(0, 3 * NQ, pass_body, 0)


@functools.cache
def _edge_call():
    return functools.partial(
        pl.kernel,
        mesh=plsc.VectorSubcoreMesh(core_axis_name="c", subcore_axis_name="s"),
        compiler_params=pltpu.CompilerParams(needs_layout_passes=False,
                                             use_tc_tiling_on_sc=False),
        out_type=jax.ShapeDtypeStruct((H * NP, ROWW), jnp.float32),
        scratch_types=[
            pltpu.VMEM((K,), jnp.int32),
            pltpu.VMEM((K,), jnp.int32),
            pltpu.VMEM((K,), jnp.int32),
            pltpu.VMEM((K,), jnp.int32),
            pltpu.VMEM((K,), jnp.int32),
            pltpu.VMEM((K,), jnp.int32),
            pltpu.VMEM((K, ROWW), jnp.float32),
            pltpu.VMEM((K, ROWW), jnp.float32),
            pltpu.VMEM((K, ROWW), jnp.float32),
            pltpu.VMEM((8, ROWW), jnp.float32),
            pltpu.VMEM((STAG,), jnp.int32),
            pltpu.VMEM((STAG,), jnp.int32),
            pltpu.VMEM((PECAP,), jnp.int32),
            pltpu.VMEM((PECAP,), jnp.int32),
            pltpu.VMEM((NP,), jnp.float32),
            pltpu.VMEM_SHARED((ACCR, ROWW), jnp.float32),
            pltpu.SemaphoreType.DMA,
            pltpu.SemaphoreType.DMA,
            pltpu.SemaphoreType.DMA,
            pltpu.SemaphoreType.DMA,
            pltpu.SemaphoreType.DMA,
            pltpu.SemaphoreType.DMA,
        ],
    )(_edge_body)


def kernel(x, edge_index, W0, a_src0, a_dst0, skip0, b0,
           W1, a_src1, a_dst1, skip1, b1,
           W2, a_src2, a_dst2, skip2, b2):
    ei = edge_index.astype(jnp.int32)
    srcs, dsts = ei[0], ei[1]
    pad2 = jnp.zeros((2, F), jnp.float32)
    h = jnp.concatenate([x, jnp.zeros((NP - N, D), jnp.float32)], axis=0)
    layers = [(W0, a_src0, a_dst0, skip0, b0, False),
              (W1, a_src1, a_dst1, skip1, b1, False),
              (W2, a_src2, a_dst2, skip2, b2, True)]
    for (W, a_s, a_d, sk, b, mean) in layers:
        asrc_pad = jnp.concatenate([a_s, pad2], axis=0)
        adst_pad = jnp.concatenate([a_d, pad2], axis=0)
        projS, sdst, skout = _prep_call(h, W, sk, asrc_pad, adst_pad)
        U = _edge_call()(projS.reshape(H * NP, ROWW), sdst.reshape(8 * NP),
                         srcs, dsts)
        h = _epi_call(U.reshape(H, NP, ROWW), skout, b, mean)
    return h[:N]


# speedup vs baseline: 1.2848x; 1.1605x over previous
import functools

import jax
import jax.numpy as jnp
from jax import lax
from jax.experimental import pallas as pl
from jax.experimental.pallas import tpu as pltpu
from jax.experimental.pallas import tpu_sc as plsc

N = 10000
NP = 10240
E = 320000
H = 6
F = 128
D = 768
ROWW = 144
TILE = 512
NT = NP // TILE
NSUB = 16
NCORE = 2
EPS = E // NSUB
K = 80
NCHUNK = EPS // K
RPS = NP // NSUB
NQ = 5
QROWS = NP // NQ
ACCR = QROWS + 8
RPQ = QROWS // NSUB
STAG = 4000
PECAP = EPS + NQ * 2 * K


def _prep_body(x_ref, w_ref, sk_ref, asrc_ref, adst_ref,
               projS_ref, sdst_ref, skout_ref):
    xt = x_ref[...]
    sdst_rows = []
    for h in range(H):
        wh = w_ref[:, h * F:(h + 1) * F]
        ph = jnp.dot(xt, wh, preferred_element_type=jnp.float32)
        projS_ref[h, :, 0:F] = ph
        ssrc = jnp.sum(ph * asrc_ref[h, :][None, :], axis=1, keepdims=True)
        projS_ref[h, :, F:ROWW] = jnp.concatenate(
            [ssrc, jnp.zeros((TILE, ROWW - F - 1), jnp.float32)], axis=1)
        sdst_rows.append(jnp.sum(ph * adst_ref[h, :][None, :], axis=1))
    sdst_rows += [jnp.zeros((TILE,), jnp.float32)] * 2
    sdst_ref[...] = jnp.stack(sdst_rows, axis=0)
    skout_ref[...] = jnp.dot(xt, sk_ref[...], preferred_element_type=jnp.float32)


def _prep_call(x, W, skipW, asrc_pad, adst_pad, interpret=False):
    return pl.pallas_call(
        _prep_body,
        grid=(NT,),
        in_specs=[
            pl.BlockSpec((TILE, D), lambda i: (i, 0)),
            pl.BlockSpec((D, D), lambda i: (0, 0)),
            pl.BlockSpec((D, D), lambda i: (0, 0)),
            pl.BlockSpec((8, F), lambda i: (0, 0)),
            pl.BlockSpec((8, F), lambda i: (0, 0)),
        ],
        out_specs=[
            pl.BlockSpec((H, TILE, ROWW), lambda i: (0, i, 0)),
            pl.BlockSpec((8, TILE), lambda i: (0, i)),
            pl.BlockSpec((TILE, D), lambda i: (i, 0)),
        ],
        out_shape=[
            jax.ShapeDtypeStruct((H, NP, ROWW), jnp.float32),
            jax.ShapeDtypeStruct((8, NP), jnp.float32),
            jax.ShapeDtypeStruct((NP, D), jnp.float32),
        ],
        interpret=interpret,
    )(x, W, skipW, asrc_pad, adst_pad)


def _epi_cat_body(u_ref, sk_ref, b_ref, out_ref):
    for h in range(H):
        u = u_ref[h, :, 0:F]
        dn = u_ref[h, :, F:F + 1]
        t = u / (dn + 1e-16) + sk_ref[:, h * F:(h + 1) * F] \
            + b_ref[0, h * F:(h + 1) * F][None, :]
        out_ref[:, h * F:(h + 1) * F] = jnp.where(t > 0, t, jnp.exp(jnp.minimum(t, 0.0)) - 1.0)


def _epi_mean_body(u_ref, sk_ref, b_ref, out_ref):
    acc = jnp.zeros((TILE, F), jnp.float32)
    for h in range(H):
        u = u_ref[h, :, 0:F]
        dn = u_ref[h, :, F:F + 1]
        acc = acc + u / (dn + 1e-16) + sk_ref[:, h * F:(h + 1) * F]
    out_ref[...] = acc * (1.0 / H) + b_ref[0, :][None, :]


def _epi_call(U, sk, b, mean, interpret=False):
    body = _epi_mean_body if mean else _epi_cat_body
    fout = F if mean else D
    return pl.pallas_call(
        body,
        grid=(NT,),
        in_specs=[
            pl.BlockSpec((H, TILE, ROWW), lambda i: (0, i, 0)),
            pl.BlockSpec((TILE, D), lambda i: (i, 0)),
            pl.BlockSpec((1, fout), lambda i: (0, 0)),
        ],
        out_specs=pl.BlockSpec((TILE, fout), lambda i: (i, 0)),
        out_shape=jax.ShapeDtypeStruct((NP, fout), jnp.float32),
        interpret=interpret,
    )(U, sk, b.reshape(1, fout))


def _edge_body(projS, sdst, srcs, dsts, u_out,
               idxA, idxB, dqbuf, rowsA, rowsB, zbuf, stag_s, stag_d,
               pe_src, pe_dst, sdst_buf, accA, semA, semB):
    c = lax.axis_index("c")
    s = lax.axis_index("s")
    ebase = s * EPS
    zero16 = jnp.zeros((16,), jnp.float32)
    c128 = jnp.full((16,), F, jnp.int32)
    i16 = lax.iota(jnp.int32, 16)
    for e in range(8):
        for r in range(ROWW // 16):
            zbuf[e, pl.ds(r * 16, 16)] = zero16

    def count_stage(st, carry):
        soff = pl.multiple_of(ebase + st * STAG, 8)
        pltpu.sync_copy(dsts.at[pl.ds(soff, STAG)], stag_d)

        def cbody(u, cnts):
            dv = stag_d[pl.ds(pl.multiple_of(u * 16, 16), 16)]
            bk = lax.shift_right_logical(dv, 11)
            return tuple(
                cnts[q] + jnp.max(jnp.cumsum(jnp.where(bk == q, 1, 0)))
                for q in range(NQ))
        return lax.fori_loop(0, STAG // 16, cbody, carry)

    cnts = lax.fori_loop(0, EPS // STAG, count_stage,
                         (jnp.int32(0),) * NQ)
    cps = [((cnt + 2 * K - 1) // (2 * K)) * (2 * K) for cnt in cnts]
    bases = [jnp.int32(0)]
    for q in range(NQ - 1):
        bases.append(bases[-1] + cps[q])

    def place_stage(st, carry):
        soff = pl.multiple_of(ebase + st * STAG, 8)
        pltpu.sync_copy(srcs.at[pl.ds(soff, STAG)], stag_s)
        pltpu.sync_copy(dsts.at[pl.ds(soff, STAG)], stag_d)

        def pbody(u, offs):
            uoff = pl.multiple_of(u * 16, 16)
            sv = stag_s[pl.ds(uoff, 16)]
            dv = stag_d[pl.ds(uoff, 16)]
            bk = lax.shift_right_logical(dv, 11)
            new_offs = []
            for q in range(NQ):
                m = bk == q
                cum = jnp.cumsum(jnp.where(m, 1, 0))
                pos = bases[q] + offs[q] + cum - 1
                plsc.store_scatter(pe_src, [pos], sv, mask=m)
                plsc.store_scatter(pe_dst, [pos], dv, mask=m)
                new_offs.append(offs[q] + jnp.max(cum))
            return tuple(new_offs)
        return lax.fori_loop(0, STAG // 16, pbody, carry)

    lax.fori_loop(0, EPS // STAG, place_stage, (jnp.int32(0),) * NQ)

    for q in range(NQ):
        for w in range(2 * K // 16):
            pos = bases[q] + cnts[q] + w * 16 + i16
            m = pos < bases[q] + cps[q]
            plsc.store_scatter(pe_src, [pos], jnp.zeros((16,), jnp.int32),
                               mask=m)
            plsc.store_scatter(pe_dst, [pos], jnp.full((16,), NP - 1,
                                                       jnp.int32), mask=m)
    steps = [cp // (2 * K) for cp in cps]

    def build_fire(idxbuf, rowsbuf, sem, qb, ch, hoff):
        boff = qb + ch * K
        for g in range(K // 16):
            sv = pe_src[pl.ds(boff + g * 16, 16)]
            idxbuf[pl.ds(g * 16, 16)] = sv + hoff
        return pltpu.async_copy(projS.at[idxbuf], rowsbuf, sem)

    def compute_scatter(rowsbuf, qb, qoff, ch):
        boff = qb + ch * K
        for g in range(K // 16):
            eidx = i16 + g * 16
            ssrc = plsc.load_gather(rowsbuf, [eidx, c128])
            dv = pe_dst[pl.ds(boff + g * 16, 16)]
            sdv = plsc.load_gather(sdst_buf, [dv])
            ev = ssrc + sdv
            ev = jnp.where(ev >= 0, ev, ev * 0.2)
            ex = jnp.exp(ev)
            plsc.store_scatter(rowsbuf, [eidx, c128], ex)
            dvq = dv - qoff
            dvq = jnp.where((dvq >= 0) & (dvq < QROWS), dvq,
                            jnp.full((16,), QROWS, jnp.int32))
            dqbuf[pl.ds(g * 16, 16)] = dvq
            for i in range(16):
                bro = lax.gather(
                    ex, jnp.full((16, 1), i, jnp.int32),
                    lax.GatherDimensionNumbers(
                        offset_dims=(), collapsed_slice_dims=(0,),
                        start_index_map=(0,)),
                    slice_sizes=(1,),
                    mode=lax.GatherScatterMode.PROMISE_IN_BOUNDS)
                e_abs = g * 16 + i
                for r in range(F // 16):
                    rowsbuf[e_abs, pl.ds(r * 16, 16)] = \
                        rowsbuf[e_abs, pl.ds(r * 16, 16)] * bro
        pltpu.sync_copy(rowsbuf, accA.at[dqbuf], add=True)

    def pass_body(t, carry0):
        q = t % NQ
        h = (t // NQ) * NCORE + c
        qoff = pl.multiple_of(q * QROWS, 8)
        hoff = pl.multiple_of(h * NP, 8)
        qb = bases[0]
        nsteps = steps[0]
        for qq in range(1, NQ):
            qb = jnp.where(q == qq, bases[qq], qb)
            nsteps = jnp.where(q == qq, steps[qq], nsteps)
        qb = pl.multiple_of(qb, 16)
        for k in range(RPQ // 8):
            pltpu.sync_copy(zbuf, accA.at[pl.ds(s * RPQ + k * 8, 8)])

        @pl.when(s == 0)
        def _zero_dump():
            pltpu.sync_copy(zbuf, accA.at[pl.ds(QROWS, 8)])

        pltpu.sync_copy(sdst.at[pl.ds(hoff, NP)], sdst_buf)
        plsc.subcore_barrier()

        @pl.when(nsteps > 0)
        def _pipeline():
            build_fire(idxA, rowsA, semA, qb, 0, hoff)
            build_fire(idxB, rowsB, semB, qb, 1, hoff)

            def step(jj, carry):
                pltpu.make_async_copy(projS.at[idxA], rowsA, semA).wait()
                compute_scatter(rowsA, qb, qoff, 2 * jj)

                @pl.when(jj + 1 < nsteps)
                def _pfa():
                    build_fire(idxA, rowsA, semA, qb, 2 * jj + 2, hoff)
                pltpu.make_async_copy(projS.at[idxB], rowsB, semB).wait()
                compute_scatter(rowsB, qb, qoff, 2 * jj + 1)

                @pl.when(jj + 1 < nsteps)
                def _pfb():
                    build_fire(idxB, rowsB, semB, qb, 2 * jj + 3, hoff)
                return carry
            lax.fori_loop(0, nsteps, step, 0)
        plsc.subcore_barrier()
        pltpu.sync_copy(accA.at[pl.ds(s * RPQ, RPQ)],
                        u_out.at[pl.ds(hoff + qoff + s * RPQ, RPQ)])
        plsc.subcore_barrier()
        return carry0

    lax.fori_loop(0, 3 * NQ, pass_body, 0)


@functools.cache
def _edge_call():
    return functools.partial(
        pl.kernel,
        mesh=plsc.VectorSubcoreMesh(core_axis_name="c", subcore_axis_name="s"),
        compiler_params=pltpu.CompilerParams(needs_layout_passes=False,
                                             use_tc_tiling_on_sc=False),
        out_type=jax.ShapeDtypeStruct((H * NP, ROWW), jnp.float32),
        scratch_types=[
            pltpu.VMEM((K,), jnp.int32),
            pltpu.VMEM((K,), jnp.int32),
            pltpu.VMEM((K,), jnp.int32),
            pltpu.VMEM((K, ROWW), jnp.float32),
            pltpu.VMEM((K, ROWW), jnp.float32),
            pltpu.VMEM((8, ROWW), jnp.float32),
            pltpu.VMEM((STAG,), jnp.int32),
            pltpu.VMEM((STAG,), jnp.int32),
            pltpu.VMEM((PECAP,), jnp.int32),
            pltpu.VMEM((PECAP,), jnp.int32),
            pltpu.VMEM((NP,), jnp.float32),
            pltpu.VMEM_SHARED((ACCR, ROWW), jnp.float32),
            pltpu.SemaphoreType.DMA,
            pltpu.SemaphoreType.DMA,
        ],
    )(_edge_body)


def kernel(x, edge_index, W0, a_src0, a_dst0, skip0, b0,
           W1, a_src1, a_dst1, skip1, b1,
           W2, a_src2, a_dst2, skip2, b2):
    ei = edge_index.astype(jnp.int32)
    srcs, dsts = ei[0], ei[1]
    pad2 = jnp.zeros((2, F), jnp.float32)
    h = jnp.concatenate([x, jnp.zeros((NP - N, D), jnp.float32)], axis=0)
    layers = [(W0, a_src0, a_dst0, skip0, b0, False),
              (W1, a_src1, a_dst1, skip1, b1, False),
              (W2, a_src2, a_dst2, skip2, b2, True)]
    for (W, a_s, a_d, sk, b, mean) in layers:
        asrc_pad = jnp.concatenate([a_s, pad2], axis=0)
        adst_pad = jnp.concatenate([a_d, pad2], axis=0)
        projS, sdst, skout = _prep_call(h, W, sk, asrc_pad, adst_pad)
        U = _edge_call()(projS.reshape(H * NP, ROWW), sdst.reshape(8 * NP),
                         srcs, dsts)
        h = _epi_call(U.reshape(H, NP, ROWW), skout, b, mean)
    return h[:N]


# speedup vs baseline: 1.2891x; 1.0034x over previous
import functools

import jax
import jax.numpy as jnp
from jax import lax
from jax.experimental import pallas as pl
from jax.experimental.pallas import tpu as pltpu
from jax.experimental.pallas import tpu_sc as plsc

N = 10000
NP = 10240
E = 320000
H = 6
F = 128
D = 768
ROWW = 144
TILE = 512
NT = NP // TILE
NSUB = 16
NCORE = 2
EPS = E // NSUB
K = 80
NCHUNK = EPS // K
RPS = NP // NSUB
NQ = 5
QROWS = NP // NQ
ACCR = QROWS + 8
RPQ = QROWS // NSUB
STAG = 4000
PECAP = EPS + NQ * 2 * K


def _prep_body(x_ref, w_ref, sk_ref, asrc_ref, adst_ref,
               projS_ref, sdst_ref, skout_ref):
    xt = x_ref[...]
    sdst_rows = []
    for h in range(H):
        wh = w_ref[:, h * F:(h + 1) * F]
        ph = jnp.dot(xt, wh, preferred_element_type=jnp.float32)
        projS_ref[h, :, 0:F] = ph
        ssrc = jnp.sum(ph * asrc_ref[h, :][None, :], axis=1, keepdims=True)
        projS_ref[h, :, F:ROWW] = jnp.concatenate(
            [ssrc, jnp.zeros((TILE, ROWW - F - 1), jnp.float32)], axis=1)
        sdst_rows.append(jnp.sum(ph * adst_ref[h, :][None, :], axis=1))
    sdst_rows += [jnp.zeros((TILE,), jnp.float32)] * 2
    sdst_ref[...] = jnp.stack(sdst_rows, axis=0)
    skout_ref[...] = jnp.dot(xt, sk_ref[...], preferred_element_type=jnp.float32)


def _prep_call(x, W, skipW, asrc_pad, adst_pad, interpret=False):
    return pl.pallas_call(
        _prep_body,
        grid=(NT,),
        in_specs=[
            pl.BlockSpec((TILE, D), lambda i: (i, 0)),
            pl.BlockSpec((D, D), lambda i: (0, 0)),
            pl.BlockSpec((D, D), lambda i: (0, 0)),
            pl.BlockSpec((8, F), lambda i: (0, 0)),
            pl.BlockSpec((8, F), lambda i: (0, 0)),
        ],
        out_specs=[
            pl.BlockSpec((H, TILE, ROWW), lambda i: (0, i, 0)),
            pl.BlockSpec((8, TILE), lambda i: (0, i)),
            pl.BlockSpec((TILE, D), lambda i: (i, 0)),
        ],
        out_shape=[
            jax.ShapeDtypeStruct((H, NP, ROWW), jnp.float32),
            jax.ShapeDtypeStruct((8, NP), jnp.float32),
            jax.ShapeDtypeStruct((NP, D), jnp.float32),
        ],
        interpret=interpret,
    )(x, W, skipW, asrc_pad, adst_pad)


def _epi_cat_body(u_ref, sk_ref, b_ref, out_ref):
    for h in range(H):
        u = u_ref[h, :, 0:F]
        dn = u_ref[h, :, F:F + 1]
        t = u / (dn + 1e-16) + sk_ref[:, h * F:(h + 1) * F] \
            + b_ref[0, h * F:(h + 1) * F][None, :]
        out_ref[:, h * F:(h + 1) * F] = jnp.where(t > 0, t, jnp.exp(jnp.minimum(t, 0.0)) - 1.0)


def _epi_mean_body(u_ref, sk_ref, b_ref, out_ref):
    acc = jnp.zeros((TILE, F), jnp.float32)
    for h in range(H):
        u = u_ref[h, :, 0:F]
        dn = u_ref[h, :, F:F + 1]
        acc = acc + u / (dn + 1e-16) + sk_ref[:, h * F:(h + 1) * F]
    out_ref[...] = acc * (1.0 / H) + b_ref[0, :][None, :]


def _epi_call(U, sk, b, mean, interpret=False):
    body = _epi_mean_body if mean else _epi_cat_body
    fout = F if mean else D
    return pl.pallas_call(
        body,
        grid=(NT,),
        in_specs=[
            pl.BlockSpec((H, TILE, ROWW), lambda i: (0, i, 0)),
            pl.BlockSpec((TILE, D), lambda i: (i, 0)),
            pl.BlockSpec((1, fout), lambda i: (0, 0)),
        ],
        out_specs=pl.BlockSpec((TILE, fout), lambda i: (i, 0)),
        out_shape=jax.ShapeDtypeStruct((NP, fout), jnp.float32),
        interpret=interpret,
    )(U, sk, b.reshape(1, fout))


def _edge_body(projS, sdst, srcs, dsts, u_out,
               idxA, idxB, dqbuf, rowsA, rowsB, zbuf, stag_s, stag_d,
               pe_src, pe_dst, sdst_buf, accA, semA, semB):
    c = lax.axis_index("c")
    s = lax.axis_index("s")
    ebase = s * EPS
    zero16 = jnp.zeros((16,), jnp.float32)
    c128 = jnp.full((16,), F, jnp.int32)
    i16 = lax.iota(jnp.int32, 16)
    for e in range(8):
        for r in range(ROWW // 16):
            zbuf[e, pl.ds(r * 16, 16)] = zero16

    def count_stage(st, carry):
        soff = pl.multiple_of(ebase + st * STAG, 8)
        pltpu.sync_copy(dsts.at[pl.ds(soff, STAG)], stag_d)

        def cbody(u, cnts):
            dv = stag_d[pl.ds(pl.multiple_of(u * 16, 16), 16)]
            bk = lax.shift_right_logical(dv, 11)
            return tuple(
                cnts[q] + jnp.max(jnp.cumsum(jnp.where(bk == q, 1, 0)))
                for q in range(NQ))
        return lax.fori_loop(0, STAG // 16, cbody, carry)

    cnts = lax.fori_loop(0, EPS // STAG, count_stage,
                         (jnp.int32(0),) * NQ)
    cps = [((cnt + 2 * K - 1) // (2 * K)) * (2 * K) for cnt in cnts]
    bases = [jnp.int32(0)]
    for q in range(NQ - 1):
        bases.append(bases[-1] + cps[q])

    def place_stage(st, carry):
        soff = pl.multiple_of(ebase + st * STAG, 8)
        pltpu.sync_copy(srcs.at[pl.ds(soff, STAG)], stag_s)
        pltpu.sync_copy(dsts.at[pl.ds(soff, STAG)], stag_d)

        def pbody(u, offs):
            uoff = pl.multiple_of(u * 16, 16)
            sv = stag_s[pl.ds(uoff, 16)]
            dv = stag_d[pl.ds(uoff, 16)]
            bk = lax.shift_right_logical(dv, 11)
            new_offs = []
            for q in range(NQ):
                m = bk == q
                cum = jnp.cumsum(jnp.where(m, 1, 0))
                pos = bases[q] + offs[q] + cum - 1
                plsc.store_scatter(pe_src, [pos], sv, mask=m)
                plsc.store_scatter(pe_dst, [pos], dv, mask=m)
                new_offs.append(offs[q] + jnp.max(cum))
            return tuple(new_offs)
        return lax.fori_loop(0, STAG // 16, pbody, carry)

    lax.fori_loop(0, EPS // STAG, place_stage, (jnp.int32(0),) * NQ)

    for q in range(NQ):
        for w in range(2 * K // 16):
            pos = bases[q] + cnts[q] + w * 16 + i16
            m = pos < bases[q] + cps[q]
            plsc.store_scatter(pe_src, [pos], jnp.zeros((16,), jnp.int32),
                               mask=m)
            plsc.store_scatter(pe_dst, [pos], jnp.full((16,), NP - 1,
                                                       jnp.int32), mask=m)
    steps = [cp // (2 * K) for cp in cps]

    def build_fire(idxbuf, rowsbuf, sem, qb, ch, hoff):
        boff = qb + ch * K
        for g in range(K // 16):
            sv = pe_src[pl.ds(boff + g * 16, 16)]
            idxbuf[pl.ds(g * 16, 16)] = sv + hoff
        return pltpu.async_copy(projS.at[idxbuf], rowsbuf, sem)

    def compute_scatter(rowsbuf, qb, qoff, ch):
        boff = qb + ch * K
        for g in range(K // 16):
            eidx = i16 + g * 16
            ssrc = plsc.load_gather(rowsbuf, [eidx, c128])
            dv = pe_dst[pl.ds(boff + g * 16, 16)]
            sdv = plsc.load_gather(sdst_buf, [dv])
            ev = ssrc + sdv
            ev = jnp.where(ev >= 0, ev, ev * 0.2)
            ex = jnp.exp(ev)
            plsc.store_scatter(rowsbuf, [eidx, c128], ex)
            dvq = dv - qoff
            dvq = jnp.where((dvq >= 0) & (dvq < QROWS), dvq,
                            jnp.full((16,), QROWS, jnp.int32))
            dqbuf[pl.ds(g * 16, 16)] = dvq
            for i in range(16):
                bro = lax.gather(
                    ex, jnp.full((16, 1), i, jnp.int32),
                    lax.GatherDimensionNumbers(
                        offset_dims=(), collapsed_slice_dims=(0,),
                        start_index_map=(0,)),
                    slice_sizes=(1,),
                    mode=lax.GatherScatterMode.PROMISE_IN_BOUNDS)
                e_abs = g * 16 + i
                for r in range(F // 16):
                    rowsbuf[e_abs, pl.ds(r * 16, 16)] = \
                        rowsbuf[e_abs, pl.ds(r * 16, 16)] * bro
        pltpu.sync_copy(rowsbuf, accA.at[dqbuf], add=True)

    def pass_body(t, carry0):
        q = t % NQ
        h = (t // NQ) * NCORE + c
        qoff = pl.multiple_of(q * QROWS, 8)
        hoff = pl.multiple_of(h * NP, 8)
        qb = bases[0]
        nsteps = steps[0]
        for qq in range(1, NQ):
            qb = jnp.where(q == qq, bases[qq], qb)
            nsteps = jnp.where(q == qq, steps[qq], nsteps)
        qb = pl.multiple_of(qb, 16)
        for k in range(RPQ // 8):
            pltpu.sync_copy(zbuf, accA.at[pl.ds(s * RPQ + k * 8, 8)])

        @pl.when(s == 0)
        def _zero_dump():
            pltpu.sync_copy(zbuf, accA.at[pl.ds(QROWS, 8)])

        pltpu.sync_copy(sdst.at[pl.ds(hoff, NP)], sdst_buf)
        plsc.subcore_barrier()

        @pl.when(nsteps > 0)
        def _pipeline():
            build_fire(idxA, rowsA, semA, qb, 0, hoff)

            def step(jj, carry):
                pltpu.make_async_copy(projS.at[idxA], rowsA, semA).wait()
                build_fire(idxB, rowsB, semB, qb, 2 * jj + 1, hoff)
                compute_scatter(rowsA, qb, qoff, 2 * jj)
                pltpu.make_async_copy(projS.at[idxB], rowsB, semB).wait()

                @pl.when(jj + 1 < nsteps)
                def _prefetch():
                    build_fire(idxA, rowsA, semA, qb, 2 * jj + 2, hoff)
                compute_scatter(rowsB, qb, qoff, 2 * jj + 1)
                return carry
            lax.fori_loop(0, nsteps, step, 0)
        plsc.subcore_barrier()
        pltpu.sync_copy(accA.at[pl.ds(s * RPQ, RPQ)],
                        u_out.at[pl.ds(hoff + qoff + s * RPQ, RPQ)])
        plsc.subcore_barrier()
        return carry0

    lax.fori_loop(0, 3 * NQ, pass_body, 0)


@functools.cache
def _edge_call():
    return functools.partial(
        pl.kernel,
        mesh=plsc.VectorSubcoreMesh(core_axis_name="c", subcore_axis_name="s"),
        compiler_params=pltpu.CompilerParams(needs_layout_passes=False,
                                             use_tc_tiling_on_sc=False),
        out_type=jax.ShapeDtypeStruct((H * NP, ROWW), jnp.float32),
        scratch_types=[
            pltpu.VMEM((K,), jnp.int32),
            pltpu.VMEM((K,), jnp.int32),
            pltpu.VMEM((K,), jnp.int32),
            pltpu.VMEM((K, ROWW), jnp.float32),
            pltpu.VMEM((K, ROWW), jnp.float32),
            pltpu.VMEM((8, ROWW), jnp.float32),
            pltpu.VMEM((STAG,), jnp.int32),
            pltpu.VMEM((STAG,), jnp.int32),
            pltpu.VMEM((PECAP,), jnp.int32),
            pltpu.VMEM((PECAP,), jnp.int32),
            pltpu.VMEM((NP,), jnp.float32),
            pltpu.VMEM_SHARED((ACCR, ROWW), jnp.float32),
            pltpu.SemaphoreType.DMA,
            pltpu.SemaphoreType.DMA,
        ],
    )(_edge_body)


def kernel(x, edge_index, W0, a_src0, a_dst0, skip0, b0,
           W1, a_src1, a_dst1, skip1, b1,
           W2, a_src2, a_dst2, skip2, b2):
    ei = edge_index.astype(jnp.int32)
    srcs, dsts = ei[0], ei[1]
    pad2 = jnp.zeros((2, F), jnp.float32)
    h = jnp.concatenate([x, jnp.zeros((NP - N, D), jnp.float32)], axis=0)
    layers = [(W0, a_src0, a_dst0, skip0, b0, False),
              (W1, a_src1, a_dst1, skip1, b1, False),
              (W2, a_src2, a_dst2, skip2, b2, True)]
    for (W, a_s, a_d, sk, b, mean) in layers:
        asrc_pad = jnp.concatenate([a_s, pad2], axis=0)
        adst_pad = jnp.concatenate([a_d, pad2], axis=0)
        projS, sdst, skout = _prep_call(h, W, sk, asrc_pad, adst_pad)
        U = _edge_call()(projS.reshape(H * NP, ROWW), sdst.reshape(8 * NP),
                         srcs, dsts)
        h = _epi_call(U.reshape(H, NP, ROWW), skout, b, mean)
    return h[:N]
